# baseline (speedup 1.0000x reference)
import functools

import jax
import jax.numpy as jnp
from jax import lax
from jax.experimental import pallas as pl
from jax.experimental.pallas import tpu as pltpu
from jax.experimental.pallas import tpu_sc as plsc

_N = 1048576
_FEAT = 64
_CPAD = 128
_CLS = 10000
_CLS_PAD = 10240

_NC = 2
_NS = 16
_NW = _NC * _NS
_HALF = _N // 2
_SPT = _HALF // _NW
_CHUNK = 128
_NCHUNKS = _SPT // _CHUNK
_HLBL = _N // _NW
_HCHUNK = 4096

_mesh = plsc.VectorSubcoreMesh(core_axis_name="c", subcore_axis_name="s")
_cparams = pltpu.CompilerParams(needs_layout_passes=False,
                                disable_bounds_checks=True)


@functools.partial(
    pl.kernel,
    out_type=jax.ShapeDtypeStruct((_NC * _CLS_PAD,), jnp.float32),
    mesh=_mesh,
    compiler_params=_cparams,
    scratch_types=[
        pltpu.VMEM((_CLS_PAD,), jnp.float32),
        pltpu.VMEM((_HCHUNK,), jnp.int32),
        pltpu.VMEM((_CLS_PAD,), jnp.float32),
        pltpu.VMEM_SHARED((16 * _CLS_PAD,), jnp.float32),
    ],
)
def _sc_bincount(labels_hbm, out_hbm, hist_v, lblbuf_v, tmp_v, slab_sp):
    c_idx = lax.axis_index("c")
    s_idx = lax.axis_index("s")
    gid = c_idx * _NS + s_idx
    iota16 = lax.iota(jnp.int32, 16)
    ones16 = jnp.ones((16,), jnp.float32)
    zeros16 = jnp.zeros((16,), jnp.float32)
    del iota16

    def zero_body(i, _):
        hist_v[pl.ds(i * 16, 16)] = zeros16
        return 0
    lax.fori_loop(0, _CLS_PAD // 16, zero_body, 0)

    hist_base = gid * _HLBL

    def hist_chunk(j, _):
        pltpu.sync_copy(labels_hbm.at[pl.ds(hist_base + j * _HCHUNK,
                                            _HCHUNK)], lblbuf_v)

        def hist_g(g, _):
            lvec = lblbuf_v[pl.ds(g * 16, 16)]
            plsc.addupdate_scatter(hist_v, [lvec], ones16)
            return 0
        lax.fori_loop(0, _HCHUNK // 16, hist_g, 0)
        return 0
    lax.fori_loop(0, _HLBL // _HCHUNK, hist_chunk, 0)

    pltpu.sync_copy(hist_v, slab_sp.at[pl.ds(s_idx * _CLS_PAD, _CLS_PAD)])
    plsc.subcore_barrier()

    @pl.when(s_idx == 0)
    def _():
        def comb_row(r, _):
            pltpu.sync_copy(slab_sp.at[pl.ds(r * _CLS_PAD, _CLS_PAD)],
                            tmp_v)

            def comb_col(f, _):
                hv = hist_v[pl.ds(f * 16, 16)]
                hist_v[pl.ds(f * 16, 16)] = hv + tmp_v[pl.ds(f * 16, 16)]
                return 0
            lax.fori_loop(0, _CLS_PAD // 16, comb_col, 0)
            return 0

        lax.fori_loop(1, 16, comb_row, 0)
        pltpu.sync_copy(hist_v,
                        out_hbm.at[pl.ds(c_idx * _CLS_PAD, _CLS_PAD)])


def _make_main(half):
    @functools.partial(
        pl.kernel,
        out_type=jax.ShapeDtypeStruct((_NW,), jnp.float32),
        mesh=_mesh,
        compiler_params=_cparams,
        scratch_types=[
            pltpu.VMEM((_CLS_PAD,), jnp.float32),
            pltpu.VMEM((_CLS_PAD,), jnp.float32),
            pltpu.VMEM((_SPT,), jnp.int32),
            pltpu.VMEM((_CHUNK, _FEAT), jnp.float32),
            pltpu.VMEM((_CHUNK, _FEAT), jnp.float32),
            pltpu.VMEM((_CHUNK, _CPAD), jnp.float32),
            pltpu.VMEM((_CHUNK, _CPAD), jnp.float32),
            pltpu.VMEM((16,), jnp.float32),
            pltpu.VMEM((_NS * 16,), jnp.float32),
            pltpu.VMEM_SHARED((_NS * 16,), jnp.float32),
            pltpu.SemaphoreType.DMA,
            pltpu.SemaphoreType.DMA,
            pltpu.SemaphoreType.DMA,
            pltpu.SemaphoreType.DMA,
        ],
    )
    def sc_main(xh_hbm, labels_hbm, center_hbm, counts_hbm, out_hbm,
                hist_v, tmp_v, lbl2_v, x2a_v, x2b_v,
                crowsa_v, crowsb_v, accbuf_v, rbuf_v, rslab_sp,
                xsem0, xsem1, gsem0, gsem1):
        c_idx = lax.axis_index("c")
        s_idx = lax.axis_index("s")
        gid = c_idx * _NS + s_idx
        iota16 = lax.iota(jnp.int32, 16)

        xbufs = (x2a_v, x2b_v)
        cbufs = (crowsa_v, crowsb_v)
        xsems = (xsem0, xsem1)
        gsems = (gsem0, gsem1)

        pltpu.sync_copy(counts_hbm.at[pl.ds(0, _CLS_PAD)], hist_v)
        pltpu.sync_copy(counts_hbm.at[pl.ds(_CLS_PAD, _CLS_PAD)], tmp_v)

        def addh(f, _):
            hv = hist_v[pl.ds(f * 16, 16)]
            hist_v[pl.ds(f * 16, 16)] = hv + tmp_v[pl.ds(f * 16, 16)]
            return 0
        lax.fori_loop(0, _CLS_PAD // 16, addh, 0)

        sbase = gid * _SPT
        pltpu.sync_copy(labels_hbm.at[pl.ds(half * _HALF + sbase, _SPT)],
                        lbl2_v)

        def wait_slot(b):
            pltpu.make_async_copy(center_hbm.at[pl.ds(0, _CHUNK)],
                                  cbufs[b], gsems[b]).wait()
            pltpu.make_async_copy(xh_hbm.at[pl.ds(0, _CHUNK)],
                                  xbufs[b], xsems[b]).wait()

        def issue(cl, b):
            pltpu.async_copy(
                center_hbm.at[lbl2_v.at[pl.ds(cl * _CHUNK, _CHUNK)]],
                cbufs[b], gsems[b])
            pltpu.async_copy(
                xh_hbm.at[pl.ds(sbase + cl * _CHUNK, _CHUNK)],
                xbufs[b], xsems[b])

        def compute(cl, b, acc):
            def group_body(g, acc):
                lvec = lbl2_v[pl.ds(cl * _CHUNK + g * 16, 16)]
                cnt = plsc.load_gather(hist_v, [lvec])
                rows = iota16 + g * 16
                facc = jnp.zeros((16,), jnp.float32)
                for p in range(_FEAT):
                    colp = (iota16 + p) & (_FEAT - 1)
                    xf = plsc.load_gather(xbufs[b], [rows, colp])
                    cf = plsc.load_gather(cbufs[b], [rows, colp])
                    d = xf - cf
                    facc = facc + d * d
                return acc + facc / cnt
            return lax.fori_loop(0, _CHUNK // 16, group_body, acc)

        issue(0, 0)

        def pair_body(p, acc):
            issue(2 * p + 1, 1)
            wait_slot(0)
            acc = compute(2 * p, 0, acc)

            @pl.when(p < _NCHUNKS // 2 - 1)
            def _():
                issue(2 * p + 2, 0)
            wait_slot(1)
            acc = compute(2 * p + 1, 1, acc)
            return acc

        acc = lax.fori_loop(0, _NCHUNKS // 2, pair_body,
                            jnp.zeros((16,), jnp.float32))

        accbuf_v[...] = acc
        pltpu.sync_copy(accbuf_v, rslab_sp.at[pl.ds(s_idx * 16, 16)])
        plsc.subcore_barrier()

        @pl.when(s_idx == 0)
        def _():
            pltpu.sync_copy(rslab_sp, rbuf_v)
            tot = rbuf_v[pl.ds(0, 16)]
            for r in range(1, 16):
                tot = tot + rbuf_v[pl.ds(r * 16, 16)]
            accbuf_v[...] = tot
            pltpu.sync_copy(accbuf_v, out_hbm.at[pl.ds(c_idx * 16, 16)])

    return sc_main


_sc_main0 = _make_main(0)
_sc_main1 = _make_main(1)


def _tc_sum_kernel(a_ref, b_ref, o_ref):
    o_ref[0, 0] = jnp.sum(a_ref[...]) + jnp.sum(b_ref[...])


def kernel(x, labels, center):
    centerp = jnp.pad(center, ((0, 0), (0, _CPAD - _FEAT)))
    counts = _sc_bincount(labels)
    p0 = _sc_main0(x[:_HALF], labels, centerp, counts)
    p1 = _sc_main1(x[_HALF:], labels, centerp, counts)
    out = pl.pallas_call(
        _tc_sum_kernel,
        out_shape=jax.ShapeDtypeStruct((1, 1), jnp.float32),
        out_specs=pl.BlockSpec(memory_space=pltpu.SMEM),
    )(p0, p1)
    return out[0, 0]


# speedup vs baseline: 1.0207x; 1.0207x over previous
import functools

import jax
import jax.numpy as jnp
from jax import lax
from jax.experimental import pallas as pl
from jax.experimental.pallas import tpu as pltpu
from jax.experimental.pallas import tpu_sc as plsc

_N = 1048576
_FEAT = 64
_CPAD = 128
_CLS = 10000
_CLS_PAD = 10240

_NC = 2
_NS = 16
_NW = _NC * _NS
_NSPLIT = 4
_PIECE = _N // _NSPLIT
_SPT = _PIECE // _NW
_CHUNK = 128
_NCHUNKS = _SPT // _CHUNK
_HLBL = _N // _NW
_HCHUNK = 4096

_mesh = plsc.VectorSubcoreMesh(core_axis_name="c", subcore_axis_name="s")
_cparams = pltpu.CompilerParams(needs_layout_passes=False,
                                disable_bounds_checks=True)


@functools.partial(
    pl.kernel,
    out_type=jax.ShapeDtypeStruct((_NC * _CLS_PAD,), jnp.float32),
    mesh=_mesh,
    compiler_params=_cparams,
    scratch_types=[
        pltpu.VMEM((_CLS_PAD,), jnp.float32),
        pltpu.VMEM((_HCHUNK,), jnp.int32),
        pltpu.VMEM((_CLS_PAD,), jnp.float32),
        pltpu.VMEM_SHARED((16 * _CLS_PAD,), jnp.float32),
    ],
)
def _sc_bincount(labels_hbm, out_hbm, hist_v, lblbuf_v, tmp_v, slab_sp):
    c_idx = lax.axis_index("c")
    s_idx = lax.axis_index("s")
    gid = c_idx * _NS + s_idx
    iota16 = lax.iota(jnp.int32, 16)
    ones16 = jnp.ones((16,), jnp.float32)
    zeros16 = jnp.zeros((16,), jnp.float32)
    del iota16

    def zero_body(i, _):
        hist_v[pl.ds(i * 16, 16)] = zeros16
        return 0
    lax.fori_loop(0, _CLS_PAD // 16, zero_body, 0)

    hist_base = gid * _HLBL

    def hist_chunk(j, _):
        pltpu.sync_copy(labels_hbm.at[pl.ds(hist_base + j * _HCHUNK,
                                            _HCHUNK)], lblbuf_v)

        def hist_g(g, _):
            lvec = lblbuf_v[pl.ds(g * 16, 16)]
            plsc.addupdate_scatter(hist_v, [lvec], ones16)
            return 0
        lax.fori_loop(0, _HCHUNK // 16, hist_g, 0)
        return 0
    lax.fori_loop(0, _HLBL // _HCHUNK, hist_chunk, 0)

    pltpu.sync_copy(hist_v, slab_sp.at[pl.ds(s_idx * _CLS_PAD, _CLS_PAD)])
    plsc.subcore_barrier()

    @pl.when(s_idx == 0)
    def _():
        def comb_row(r, _):
            pltpu.sync_copy(slab_sp.at[pl.ds(r * _CLS_PAD, _CLS_PAD)],
                            tmp_v)

            def comb_col(f, _):
                hv = hist_v[pl.ds(f * 16, 16)]
                hist_v[pl.ds(f * 16, 16)] = hv + tmp_v[pl.ds(f * 16, 16)]
                return 0
            lax.fori_loop(0, _CLS_PAD // 16, comb_col, 0)
            return 0

        lax.fori_loop(1, 16, comb_row, 0)
        pltpu.sync_copy(hist_v,
                        out_hbm.at[pl.ds(c_idx * _CLS_PAD, _CLS_PAD)])


def _make_main(piece):
    @functools.partial(
        pl.kernel,
        out_type=jax.ShapeDtypeStruct((_NW,), jnp.float32),
        mesh=_mesh,
        compiler_params=_cparams,
        scratch_types=[
            pltpu.VMEM((_CLS_PAD,), jnp.float32),
            pltpu.VMEM((_CLS_PAD,), jnp.float32),
            pltpu.VMEM((_SPT,), jnp.int32),
            pltpu.VMEM((_CHUNK, _FEAT), jnp.float32),
            pltpu.VMEM((_CHUNK, _FEAT), jnp.float32),
            pltpu.VMEM((_CHUNK, _CPAD), jnp.float32),
            pltpu.VMEM((_CHUNK, _CPAD), jnp.float32),
            pltpu.VMEM((16,), jnp.float32),
            pltpu.VMEM((_NS * 16,), jnp.float32),
            pltpu.VMEM_SHARED((_NS * 16,), jnp.float32),
            pltpu.SemaphoreType.DMA,
            pltpu.SemaphoreType.DMA,
            pltpu.SemaphoreType.DMA,
            pltpu.SemaphoreType.DMA,
        ],
    )
    def sc_main(xh_hbm, labels_hbm, center_hbm, counts_hbm, out_hbm,
                hist_v, tmp_v, lbl2_v, x2a_v, x2b_v,
                crowsa_v, crowsb_v, accbuf_v, rbuf_v, rslab_sp,
                xsem0, xsem1, gsem0, gsem1):
        c_idx = lax.axis_index("c")
        s_idx = lax.axis_index("s")
        gid = c_idx * _NS + s_idx
        iota16 = lax.iota(jnp.int32, 16)

        xbufs = (x2a_v, x2b_v)
        cbufs = (crowsa_v, crowsb_v)
        xsems = (xsem0, xsem1)
        gsems = (gsem0, gsem1)

        pltpu.sync_copy(counts_hbm.at[pl.ds(0, _CLS_PAD)], hist_v)
        pltpu.sync_copy(counts_hbm.at[pl.ds(_CLS_PAD, _CLS_PAD)], tmp_v)

        def addh(f, _):
            hv = hist_v[pl.ds(f * 16, 16)]
            hist_v[pl.ds(f * 16, 16)] = hv + tmp_v[pl.ds(f * 16, 16)]
            return 0
        lax.fori_loop(0, _CLS_PAD // 16, addh, 0)

        sbase = gid * _SPT
        pltpu.sync_copy(labels_hbm.at[pl.ds(piece * _PIECE + sbase, _SPT)],
                        lbl2_v)

        def wait_slot(b):
            pltpu.make_async_copy(center_hbm.at[pl.ds(0, _CHUNK)],
                                  cbufs[b], gsems[b]).wait()
            pltpu.make_async_copy(xh_hbm.at[pl.ds(0, _CHUNK)],
                                  xbufs[b], xsems[b]).wait()

        def issue(cl, b):
            pltpu.async_copy(
                center_hbm.at[lbl2_v.at[pl.ds(cl * _CHUNK, _CHUNK)]],
                cbufs[b], gsems[b])
            pltpu.async_copy(
                xh_hbm.at[pl.ds(sbase + cl * _CHUNK, _CHUNK)],
                xbufs[b], xsems[b])

        def compute(cl, b, acc):
            def group_body(g, acc):
                lvec = lbl2_v[pl.ds(cl * _CHUNK + g * 16, 16)]
                cnt = plsc.load_gather(hist_v, [lvec])
                rows = iota16 + g * 16
                facc = jnp.zeros((16,), jnp.float32)
                for p in range(_FEAT):
                    colp = (iota16 + p) & (_FEAT - 1)
                    xf = plsc.load_gather(xbufs[b], [rows, colp])
                    cf = plsc.load_gather(cbufs[b], [rows, colp])
                    d = xf - cf
                    facc = facc + d * d
                return acc + facc / cnt
            return lax.fori_loop(0, _CHUNK // 16, group_body, acc)

        issue(0, 0)

        def pair_body(p, acc):
            issue(2 * p + 1, 1)
            wait_slot(0)
            acc = compute(2 * p, 0, acc)

            @pl.when(p < _NCHUNKS // 2 - 1)
            def _():
                issue(2 * p + 2, 0)
            wait_slot(1)
            acc = compute(2 * p + 1, 1, acc)
            return acc

        acc = lax.fori_loop(0, _NCHUNKS // 2, pair_body,
                            jnp.zeros((16,), jnp.float32))

        accbuf_v[...] = acc
        pltpu.sync_copy(accbuf_v, rslab_sp.at[pl.ds(s_idx * 16, 16)])
        plsc.subcore_barrier()

        @pl.when(s_idx == 0)
        def _():
            pltpu.sync_copy(rslab_sp, rbuf_v)
            tot = rbuf_v[pl.ds(0, 16)]
            for r in range(1, 16):
                tot = tot + rbuf_v[pl.ds(r * 16, 16)]
            accbuf_v[...] = tot
            pltpu.sync_copy(accbuf_v, out_hbm.at[pl.ds(c_idx * 16, 16)])

    return sc_main


_sc_mains = [_make_main(q) for q in range(_NSPLIT)]


def _tc_sum_kernel(*refs):
    o_ref = refs[-1]
    o_ref[0, 0] = sum(jnp.sum(r[...]) for r in refs[:-1])


def kernel(x, labels, center):
    centerp = jnp.pad(center, ((0, 0), (0, _CPAD - _FEAT)))
    counts = _sc_bincount(labels)
    parts = [
        _sc_mains[q](x[q * _PIECE:(q + 1) * _PIECE], labels, centerp,
                     counts)
        for q in range(_NSPLIT)
    ]
    out = pl.pallas_call(
        _tc_sum_kernel,
        out_shape=jax.ShapeDtypeStruct((1, 1), jnp.float32),
        out_specs=pl.BlockSpec(memory_space=pltpu.SMEM),
    )(*parts)
    return out[0, 0]


# speedup vs baseline: 1.0261x; 1.0052x over previous
import functools

import jax
import jax.numpy as jnp
from jax import lax
from jax.experimental import pallas as pl
from jax.experimental.pallas import tpu as pltpu
from jax.experimental.pallas import tpu_sc as plsc

_N = 1048576
_FEAT = 64
_CPAD = 128
_CLS = 10000
_CLS_PAD = 10240

_NC = 2
_NS = 16
_SAMPLES_PER_TILE = _N // (_NC * _NS)
_CHUNK = 128
_NCHUNKS = _SAMPLES_PER_TILE // _CHUNK
_HLBL = _N // _NS
_HCHUNK = 4096


def _sc_center_loss(x1d, labels1, centerp):
    mesh = plsc.VectorSubcoreMesh(core_axis_name="c", subcore_axis_name="s")

    @functools.partial(
        pl.kernel,
        out_type=jax.ShapeDtypeStruct((_NC * 16,), jnp.float32),
        mesh=mesh,
        compiler_params=pltpu.CompilerParams(
            needs_layout_passes=False, disable_bounds_checks=True),
        scratch_types=[
            pltpu.VMEM((_CLS_PAD,), jnp.float32),
            pltpu.VMEM((_HCHUNK,), jnp.int32),
            pltpu.VMEM((_CLS_PAD,), jnp.float32),
            pltpu.VMEM((_NCHUNKS // 2 * _CHUNK,), jnp.int32),
            pltpu.VMEM((_CHUNK, _FEAT), jnp.float32),
            pltpu.VMEM((_CHUNK, _FEAT), jnp.float32),
            pltpu.VMEM((_CHUNK, _CPAD), jnp.float32),
            pltpu.VMEM((_CHUNK, _CPAD), jnp.float32),
            pltpu.VMEM((16,), jnp.float32),
            pltpu.VMEM((_NS * 16,), jnp.float32),
            pltpu.VMEM_SHARED((16 * _CLS_PAD,), jnp.float32),
            pltpu.VMEM_SHARED((_NS * 16,), jnp.float32),
            pltpu.SemaphoreType.DMA,
            pltpu.SemaphoreType.DMA,
            pltpu.SemaphoreType.DMA,
            pltpu.SemaphoreType.DMA,
        ],
    )
    def sc_kernel(x_hbm, labels_hbm, center_hbm, out_hbm,
                  hist_v, lblbuf_v, tmp_v, lbl2_v, x2a_v, x2b_v,
                  crowsa_v, crowsb_v, accbuf_v, rbuf_v, slab_sp, rslab_sp,
                  xsem0, xsem1, gsem0, gsem1):
        c_idx = lax.axis_index("c")
        s_idx = lax.axis_index("s")
        gid = c_idx * _NS + s_idx
        iota16 = lax.iota(jnp.int32, 16)
        ones16 = jnp.ones((16,), jnp.float32)
        zeros16 = jnp.zeros((16,), jnp.float32)

        xbufs = (x2a_v, x2b_v)
        cbufs = (crowsa_v, crowsb_v)
        xsems = (xsem0, xsem1)
        gsems = (gsem0, gsem1)

        def zero_body(i, _):
            hist_v[pl.ds(i * 16, 16)] = zeros16
            return 0
        lax.fori_loop(0, _CLS_PAD // 16, zero_body, 0)

        hist_base = s_idx * _HLBL

        def hist_chunk(j, _):
            pltpu.sync_copy(labels_hbm.at[pl.ds(hist_base + j * _HCHUNK,
                                                _HCHUNK)], lblbuf_v)

            def hist_g(g, _):
                lvec = lblbuf_v[pl.ds(g * 16, 16)]
                plsc.addupdate_scatter(hist_v, [lvec], ones16)
                return 0
            lax.fori_loop(0, _HCHUNK // 16, hist_g, 0)
            return 0
        lax.fori_loop(0, _HLBL // _HCHUNK, hist_chunk, 0)

        pltpu.sync_copy(hist_v, slab_sp.at[pl.ds(s_idx * _CLS_PAD,
                                                 _CLS_PAD)])
        plsc.subcore_barrier()

        def comb_chunk(cb, _):
            def comb_row(r, _):
                pltpu.sync_copy(
                    slab_sp.at[pl.ds(r * _CLS_PAD + cb * 640, 640)],
                    tmp_v.at[pl.ds(r * 640, 640)])
                return 0
            lax.fori_loop(0, 16, comb_row, 0)

            def comb_col(f, _):
                acc = tmp_v[pl.ds(f * 16, 16)]
                for r in range(1, 16):
                    acc = acc + tmp_v[pl.ds(r * 640 + f * 16, 16)]
                hist_v[pl.ds(cb * 640 + f * 16, 16)] = acc
                return 0
            lax.fori_loop(0, 40, comb_col, 0)
            return 0
        lax.fori_loop(0, 16, comb_chunk, 0)

        main_base = gid * _SAMPLES_PER_TILE
        halfsamp = _SAMPLES_PER_TILE // 2
        halfchunks = _NCHUNKS // 2

        def wait_slot(b):
            pltpu.make_async_copy(center_hbm.at[pl.ds(0, _CHUNK)],
                                  cbufs[b], gsems[b]).wait()
            pltpu.make_async_copy(x_hbm.at[pl.ds(0, _CHUNK)],
                                  xbufs[b], xsems[b]).wait()

        def half_body(h, acc):
            sbase = main_base + h * halfsamp
            pltpu.sync_copy(labels_hbm.at[pl.ds(sbase, halfsamp)], lbl2_v)

            def issue(cl, b):
                pltpu.async_copy(
                    center_hbm.at[lbl2_v.at[pl.ds(cl * _CHUNK, _CHUNK)]],
                    cbufs[b], gsems[b])
                pltpu.async_copy(
                    x_hbm.at[pl.ds(sbase + cl * _CHUNK, _CHUNK)],
                    xbufs[b], xsems[b])

            def compute(cl, b, acc):
                def group_body(g, acc):
                    lvec = lbl2_v[pl.ds(cl * _CHUNK + g * 16, 16)]
                    cnt = plsc.load_gather(hist_v, [lvec])
                    rows = iota16 + g * 16
                    facc = jnp.zeros((16,), jnp.float32)
                    for p in range(_FEAT):
                        colp = (iota16 + p) & (_FEAT - 1)
                        xf = plsc.load_gather(xbufs[b], [rows, colp])
                        cf = plsc.load_gather(cbufs[b], [rows, colp])
                        d = xf - cf
                        facc = facc + d * d
                    return acc + facc / cnt
                return lax.fori_loop(0, _CHUNK // 16, group_body, acc)

            issue(0, 0)

            def pair_body(p, acc):
                issue(2 * p + 1, 1)
                wait_slot(0)
                acc = compute(2 * p, 0, acc)

                @pl.when(p < halfchunks // 2 - 1)
                def _():
                    issue(2 * p + 2, 0)
                wait_slot(1)
                acc = compute(2 * p + 1, 1, acc)
                return acc

            return lax.fori_loop(0, halfchunks // 2, pair_body, acc)

        acc = lax.fori_loop(0, 2, half_body, jnp.zeros((16,), jnp.float32))

        accbuf_v[...] = acc
        pltpu.sync_copy(accbuf_v, rslab_sp.at[pl.ds(s_idx * 16, 16)])
        plsc.subcore_barrier()

        @pl.when(s_idx == 0)
        def _():
            pltpu.sync_copy(rslab_sp, rbuf_v)
            tot = rbuf_v[pl.ds(0, 16)]
            for r in range(1, 16):
                tot = tot + rbuf_v[pl.ds(r * 16, 16)]
            accbuf_v[...] = tot
            pltpu.sync_copy(accbuf_v, out_hbm.at[pl.ds(c_idx * 16, 16)])

    return sc_kernel(x1d, labels1, centerp)


def _tc_sum_kernel(in_ref, o_ref):
    o_ref[0, 0] = jnp.sum(in_ref[...])


def kernel(x, labels, center):
    centerp = jnp.pad(center, ((0, 0), (0, _CPAD - _FEAT)))
    part = _sc_center_loss(x, labels, centerp)
    out = pl.pallas_call(
        _tc_sum_kernel,
        out_shape=jax.ShapeDtypeStruct((1, 1), jnp.float32),
        out_specs=pl.BlockSpec(memory_space=pltpu.SMEM),
    )(part)
    return out[0, 0]
